# Initial kernel scaffold; baseline (speedup 1.0000x reference)
import jax
import jax.numpy as jnp
from jax.experimental import pallas as pl


def kernel(x, gate_w, gate_b, gate_temp, w1, b1, w2, b2, ws1, bs1, ws2, bs2):
    raise NotImplementedError("write your pallas kernel here")


# speedup vs baseline: 1.2664x; 1.2664x over previous
import functools

import jax
import jax.numpy as jnp
from jax.experimental import pallas as pl
from jax.experimental.pallas import tpu as pltpu

D_MODEL = 768
NUM_EXPERTS = 8
EXPERT_DIM = 2048
TOP_K = 2
EM_ITERS = 5
LOAD_BALANCE_WEIGHT = 0.1


def _router_body(x_ref, gw_ref, gb_ref, gt_ref, cw_ref, loss_ref):
    x = x_ref[...]
    T = x.shape[0]
    E = NUM_EXPERTS

    logits = jnp.dot(x, gw_ref[...], preferred_element_type=jnp.float32)
    logits = (logits + gb_ref[...]) / gt_ref[0, 0]

    m = jnp.max(logits, axis=1, keepdims=True)
    ex = jnp.exp(logits - m)
    sm = ex / jnp.sum(ex, axis=1, keepdims=True)

    iota = jax.lax.broadcasted_iota(jnp.int32, (T, E), 1)

    m1 = jnp.max(sm, axis=1, keepdims=True)
    i1 = jnp.min(jnp.where(sm == m1, iota, E), axis=1, keepdims=True)
    sm_masked = jnp.where(iota == i1, -jnp.inf, sm)
    m2 = jnp.max(sm_masked, axis=1, keepdims=True)
    i2 = jnp.min(jnp.where(sm_masked == m2, iota, E), axis=1, keepdims=True)

    oh1 = (iota == i1).astype(jnp.float32)
    oh2 = (iota == i2).astype(jnp.float32)

    usage = jnp.sum(oh1, axis=0, keepdims=True)
    actual = usage / jnp.float32(T) + 1e-8
    actual = actual / jnp.sum(actual)
    unif = jnp.float32(1.0 / E)
    kl = jnp.sum(unif * (jnp.log(unif) - jnp.log(actual)),
                 axis=1, keepdims=True)
    loss_ref[...] = LOAD_BALANCE_WEIGHT * kl

    p = jnp.full((1, E), 1.0 / E, dtype=jnp.float32)
    for _ in range(EM_ITERS):
        ea = sm * p
        ea = ea / (jnp.sum(ea, axis=1, keepdims=True) + 1e-8)
        counts = jnp.sum(ea, axis=0, keepdims=True)
        p = counts / (jnp.sum(counts) + 1e-8)

    bal1 = m1[:, 0] * jnp.sum(oh1 * p, axis=1)
    bal2 = m2[:, 0] * jnp.sum(oh2 * p, axis=1)
    denom = bal1 + bal2 + 1e-8
    c1 = bal1 / denom
    c2 = bal2 / denom

    cw_ref[...] = oh1 * c1[:, None] + oh2 * c2[:, None]


def _expert_body(x_ref, cw_ref, w1_ref, b1_ref, w2_ref, b2_ref,
                 ws1_ref, bs1_ref, ws2_ref, bs2_ref, out_ref):
    e = pl.program_id(1)
    x = x_ref[...]

    @pl.when(e == 0)
    def _():
        hs = jnp.maximum(
            jnp.dot(x, ws1_ref[...], preferred_element_type=jnp.float32)
            + bs1_ref[...], 0.0)
        out_ref[...] = (
            jnp.dot(hs, ws2_ref[...], preferred_element_type=jnp.float32)
            + bs2_ref[...])

    h = jnp.maximum(
        jnp.dot(x, w1_ref[0], preferred_element_type=jnp.float32)
        + b1_ref[0], 0.0)
    y = (jnp.dot(h, w2_ref[0], preferred_element_type=jnp.float32)
         + b2_ref[0])

    lane = jax.lax.broadcasted_iota(jnp.int32, (1, NUM_EXPERTS), 1)
    wcol = jnp.sum(cw_ref[...] * (lane == e).astype(jnp.float32),
                   axis=1, keepdims=True)
    out_ref[...] += wcol * y


@jax.jit
def kernel(x, gate_w, gate_b, gate_temp, w1, b1, w2, b2, ws1, bs1, ws2, bs2):
    B, S, D = x.shape
    T = B * S
    E = NUM_EXPERTS
    F = EXPERT_DIM
    x_flat = x.reshape(T, D)

    cw, loss = pl.pallas_call(
        _router_body,
        out_shape=(
            jax.ShapeDtypeStruct((T, E), jnp.float32),
            jax.ShapeDtypeStruct((1, 1), jnp.float32),
        ),
    )(x_flat, gate_w, gate_b.reshape(1, E), gate_temp.reshape(1, 1))

    TB = 512
    n_tb = T // TB

    out = pl.pallas_call(
        _expert_body,
        grid=(n_tb, E),
        in_specs=[
            pl.BlockSpec((TB, D), lambda t, e: (t, 0)),
            pl.BlockSpec((TB, E), lambda t, e: (t, 0)),
            pl.BlockSpec((1, D, F), lambda t, e: (e, 0, 0)),
            pl.BlockSpec((1, 1, F), lambda t, e: (e, 0, 0)),
            pl.BlockSpec((1, F, D), lambda t, e: (e, 0, 0)),
            pl.BlockSpec((1, 1, D), lambda t, e: (e, 0, 0)),
            pl.BlockSpec((D, F), lambda t, e: (0, 0)),
            pl.BlockSpec((1, F), lambda t, e: (0, 0)),
            pl.BlockSpec((F, D), lambda t, e: (0, 0)),
            pl.BlockSpec((1, D), lambda t, e: (0, 0)),
        ],
        out_specs=pl.BlockSpec((TB, D), lambda t, e: (t, 0)),
        out_shape=jax.ShapeDtypeStruct((T, D), jnp.float32),
    )(x_flat, cw, w1, b1.reshape(E, 1, F), w2, b2.reshape(E, 1, D),
      ws1, bs1.reshape(1, F), ws2, bs2.reshape(1, D))

    return out.reshape(B, S, D), loss.reshape(())
